# Initial kernel scaffold; baseline (speedup 1.0000x reference)
import jax
import jax.numpy as jnp
from jax.experimental import pallas as pl


def kernel(updates, mask):
    raise NotImplementedError("write your pallas kernel here")


# speedup vs baseline: 2.8170x; 2.8170x over previous
import functools

import jax
import jax.numpy as jnp
from jax import lax
from jax.experimental import pallas as pl
from jax.experimental.pallas import tpu as pltpu
from jax.experimental.pallas import tpu_sc as plsc

B, H, W_IN, C_CH = 8, 112, 112, 96
M = B * H * W_IN * C_CH
N = M * 4
NC, NS = 2, 16
NP = 14
WWIN = N // (NP * NC)
SEG = M // NS
CHUNK = 6144
NCHUNK = SEG // CHUNK
ROWS = CHUNK // 128
FLUSH = WWIN // NS
ZCH = 6144
NZ = FLUSH // ZCH


def _body(idx_hbm, upd_hbm, out_hbm, win_ref, idx_in, upd_in, cidx, cval, zbuf):
    cid = lax.axis_index("c")
    sid = lax.axis_index("s")
    zeros16 = jnp.zeros((16,), jnp.float32)
    iota16 = lax.iota(jnp.int32, 16)
    fbbase = iota16 * 416

    def zero_zbuf(i, _):
        zbuf[pl.ds(i * 16, 16)] = zeros16
        return 0

    lax.fori_loop(0, ZCH // 16, zero_zbuf, 0)

    def one_pass(p, _):
        win = p * NC + cid
        base = win * WWIN

        def zfill(z, _):
            pltpu.sync_copy(zbuf, win_ref.at[pl.ds(sid * FLUSH + z * ZCH, ZCH)])
            return 0

        lax.fori_loop(0, NZ, zfill, 0)
        plsc.subcore_barrier()

        def one_chunk(ch, _):
            off = sid * SEG + ch * CHUNK
            pltpu.sync_copy(idx_hbm.at[pl.ds(off, CHUNK)], idx_in)
            pltpu.sync_copy(upd_hbm.at[pl.ds(off, CHUNK)], upd_in)

            def one_row(r, _):
                for c in range(8):
                    col = c * 16
                    iv = idx_in[pl.ds(r * 128 + col, 16)]
                    uv = upd_in[pl.ds(r * 128 + col, 16)]
                    t = iv - base
                    hit = plsc.bitcast(t, jnp.uint32) < jnp.uint32(WWIN)
                    fb = fbbase + c * 52
                    cidx[r, pl.ds(col, 16)] = jnp.where(hit, t, fb)
                    cval[r, pl.ds(col, 16)] = jnp.where(hit, uv, 0.0)
                return 0

            lax.fori_loop(0, ROWS, one_row, 0)

            def scat(r, _):
                pltpu.sync_copy(cval.at[r], win_ref.at[cidx.at[r]], add=True)
                return 0

            lax.fori_loop(0, ROWS, scat, 0)
            return 0

        lax.fori_loop(0, NCHUNK, one_chunk, 0)
        plsc.subcore_barrier()

        pltpu.sync_copy(
            win_ref.at[pl.ds(sid * FLUSH, FLUSH)],
            out_hbm.at[pl.ds(base + sid * FLUSH, FLUSH)],
        )
        plsc.subcore_barrier()
        return 0

    lax.fori_loop(0, NP, one_pass, 0)


@jax.jit
def kernel(updates, mask):
    flat_idx = jnp.reshape(mask, (-1,)).astype(jnp.int32)
    flat_upd = jnp.reshape(updates, (-1,))
    mesh = plsc.VectorSubcoreMesh(core_axis_name="c", subcore_axis_name="s")
    out = pl.kernel(
        _body,
        out_type=jax.ShapeDtypeStruct((N,), jnp.float32),
        mesh=mesh,
        scratch_types=[
            pltpu.VMEM_SHARED((WWIN,), jnp.float32),
            pltpu.VMEM((CHUNK,), jnp.int32),
            pltpu.VMEM((CHUNK,), jnp.float32),
            pltpu.VMEM((ROWS, 128), jnp.int32),
            pltpu.VMEM((ROWS, 128), jnp.float32),
            pltpu.VMEM((ZCH,), jnp.float32),
        ],
    )(flat_idx, flat_upd)
    return jnp.reshape(out, (B, H * 2, W_IN * 2, C_CH))


# speedup vs baseline: 3.6852x; 1.3082x over previous
import jax
import jax.numpy as jnp
from jax import lax
from jax.experimental import pallas as pl
from jax.experimental.pallas import tpu as pltpu
from jax.experimental.pallas import tpu_sc as plsc

B, H, W_IN, C_CH = 8, 112, 112, 96
M = B * H * W_IN * C_CH
N = M * 4
NC, NS = 2, 16
NP = 14
WWIN = N // (NP * NC)
SEG = M // NS
CHUNK = 6144
NCHUNK = SEG // CHUNK
ROWS = CHUNK // 128
RC = 64
FLUSH = WWIN // NS
ZCH = 2048
NZ = FLUSH // ZCH


def _body(idx_hbm, upd_hbm, out_hbm, win_ref, i0, v0, i1, v1, cidx, cval,
          zbuf, sem0, sem1):
    cid = lax.axis_index("c")
    sid = lax.axis_index("s")
    zeros16 = jnp.zeros((16,), jnp.float32)
    iota16 = lax.iota(jnp.int32, 16)
    fbbase = iota16 * 416

    def zero_zbuf(i, _):
        zbuf[pl.ds(i * 16, 16)] = zeros16
        return 0

    lax.fori_loop(0, ZCH // 16, zero_zbuf, 0)

    def fire(ch, ib, vb, sem):
        off = sid * SEG + ch * CHUNK
        pltpu.async_copy(idx_hbm.at[pl.ds(off, CHUNK)], ib, sem)
        pltpu.async_copy(upd_hbm.at[pl.ds(off, CHUNK)], vb, sem)

    def wait(ib, vb, sem):
        pltpu.make_async_copy(idx_hbm.at[pl.ds(0, CHUNK)], ib, sem).wait()
        pltpu.make_async_copy(upd_hbm.at[pl.ds(0, CHUNK)], vb, sem).wait()

    def one_pass(p, _):
        win = p * NC + cid
        base = win * WWIN

        def zfill(z, _):
            pltpu.sync_copy(zbuf, win_ref.at[pl.ds(sid * FLUSH + z * ZCH, ZCH)])
            return 0

        lax.fori_loop(0, NZ, zfill, 0)
        plsc.subcore_barrier()

        def process(ib, vb, cnt, flushed):
            def one_row(r, cnt):
                for c in range(8):
                    sl = pl.ds(r * 128 + c * 16, 16)
                    t = ib[sl] - base
                    uv = vb[sl]
                    hit = plsc.bitcast(t, jnp.uint32) < jnp.uint32(WWIN)
                    off = plsc.cumsum(jnp.where(hit, 1, 0))
                    pos = cnt + off - 1
                    row = jnp.bitwise_and(
                        lax.shift_right_logical(pos, 7), RC - 1)
                    col = jnp.bitwise_and(pos, 127)
                    plsc.store_scatter(cidx, [row, col], t, mask=hit)
                    plsc.store_scatter(cval, [row, col], uv, mask=hit)
                    cnt = cnt + plsc.all_reduce_population_count(hit)
                return cnt

            cnt = lax.fori_loop(0, ROWS, one_row, cnt)
            full = lax.shift_right_logical(jnp.max(cnt), 7)

            def flush_row(s, _):
                r = jnp.bitwise_and(s, RC - 1)
                pltpu.sync_copy(cval.at[r], win_ref.at[cidx.at[r]], add=True)
                return 0

            lax.fori_loop(flushed, full, flush_row, 0)
            return cnt, full

        def two_chunks(g, carry):
            cnt, flushed = carry
            fire(2 * g + 1, i1, v1, sem1)
            wait(i0, v0, sem0)
            cnt, flushed = process(i0, v0, cnt, flushed)

            @pl.when(g < NCHUNK // 2 - 1)
            def _():
                fire(2 * g + 2, i0, v0, sem0)

            wait(i1, v1, sem1)
            return process(i1, v1, cnt, flushed)

        fire(0, i0, v0, sem0)
        cnt, _ = lax.fori_loop(
            0, NCHUNK // 2, two_chunks,
            (jnp.zeros((16,), jnp.int32), jnp.int32(0)))

        cnt_s = jnp.max(cnt)
        q = jnp.bitwise_and(cnt_s, 127)
        prow = jnp.bitwise_and(lax.shift_right_logical(cnt_s, 7), RC - 1)

        @pl.when(q > 0)
        def _():
            for j in range(8):
                slc = pl.ds(j * 16, 16)
                keep = (iota16 + j * 16) < q
                cval[prow, slc] = jnp.where(keep, cval[prow, slc], 0.0)
                cidx[prow, slc] = jnp.where(keep, cidx[prow, slc],
                                            fbbase + j * 52)
            pltpu.sync_copy(cval.at[prow], win_ref.at[cidx.at[prow]],
                            add=True)

        plsc.subcore_barrier()

        pltpu.sync_copy(
            win_ref.at[pl.ds(sid * FLUSH, FLUSH)],
            out_hbm.at[pl.ds(base + sid * FLUSH, FLUSH)],
        )
        plsc.subcore_barrier()
        return 0

    lax.fori_loop(0, NP, one_pass, 0)


@jax.jit
def kernel(updates, mask):
    flat_idx = jnp.reshape(mask, (-1,)).astype(jnp.int32)
    flat_upd = jnp.reshape(updates, (-1,))
    mesh = plsc.VectorSubcoreMesh(core_axis_name="c", subcore_axis_name="s")
    out = pl.kernel(
        _body,
        compiler_params=pltpu.CompilerParams(needs_layout_passes=False),
        out_type=jax.ShapeDtypeStruct((N,), jnp.float32),
        mesh=mesh,
        scratch_types=[
            pltpu.VMEM_SHARED((WWIN,), jnp.float32),
            pltpu.VMEM((CHUNK,), jnp.int32),
            pltpu.VMEM((CHUNK,), jnp.float32),
            pltpu.VMEM((CHUNK,), jnp.int32),
            pltpu.VMEM((CHUNK,), jnp.float32),
            pltpu.VMEM((RC, 128), jnp.int32),
            pltpu.VMEM((RC, 128), jnp.float32),
            pltpu.VMEM((ZCH,), jnp.float32),
            pltpu.SemaphoreType.DMA,
            pltpu.SemaphoreType.DMA,
        ],
    )(flat_idx, flat_upd)
    return jnp.reshape(out, (B, H * 2, W_IN * 2, C_CH))


# speedup vs baseline: 3.8163x; 1.0356x over previous
import jax
import jax.numpy as jnp
from jax import lax
from jax.experimental import pallas as pl
from jax.experimental.pallas import tpu as pltpu
from jax.experimental.pallas import tpu_sc as plsc

B, H, W_IN, C_CH = 8, 112, 112, 96
M = B * H * W_IN * C_CH
N = M * 4
NC, NS = 2, 16
NP = 14
WWIN = N // (NP * NC)
SEG = M // NS
CHUNK = 6144
NCHUNK = SEG // CHUNK
ROWS = CHUNK // 128
RW = 128
RC = 64
MAXOUT = 8
FLUSH = WWIN // NS
ZCH = 2048
NZ = FLUSH // ZCH


def _body(idx_hbm, upd_hbm, out_hbm, win_ref, i0, v0, i1, v1, cidx, cval,
          zbuf, sem0, sem1, semz, sems):
    cid = lax.axis_index("c")
    sid = lax.axis_index("s")
    zeros16 = jnp.zeros((16,), jnp.float32)
    iota16 = lax.iota(jnp.int32, 16)

    def zero_zbuf(i, _):
        zbuf[pl.ds(i * 16, 16)] = zeros16
        return 0

    lax.fori_loop(0, ZCH // 16, zero_zbuf, 0)

    def fire(ch, ib, vb, sem):
        off = sid * SEG + ch * CHUNK
        pltpu.async_copy(idx_hbm.at[pl.ds(off, CHUNK)], ib, sem)
        pltpu.async_copy(upd_hbm.at[pl.ds(off, CHUNK)], vb, sem)

    def wait(ib, vb, sem):
        pltpu.make_async_copy(idx_hbm.at[pl.ds(0, CHUNK)], ib, sem).wait()
        pltpu.make_async_copy(upd_hbm.at[pl.ds(0, CHUNK)], vb, sem).wait()

    def wait_scat():
        pltpu.make_async_copy(cval.at[0], win_ref.at[cidx.at[0]],
                              sems).wait()

    def one_pass(p, _):
        win = p * NC + cid
        base = win * WWIN

        fire(0, i0, v0, sem0)
        fire(1, i1, v1, sem1)

        def zfill(z, _):
            pltpu.async_copy(
                zbuf, win_ref.at[pl.ds(sid * FLUSH + z * ZCH, ZCH)], semz)
            return 0

        lax.fori_loop(0, NZ, zfill, 0)

        def zwait(z, _):
            pltpu.make_async_copy(
                zbuf, win_ref.at[pl.ds(sid * FLUSH, ZCH)], semz).wait()
            return 0

        lax.fori_loop(0, NZ, zwait, 0)
        plsc.subcore_barrier()

        def process(ib, vb, cnt, flushed, waited):
            def one_row(r, cnt):
                for c in range(8):
                    sl = pl.ds(r * 128 + c * 16, 16)
                    t = ib[sl] - base
                    uv = vb[sl]
                    hit = plsc.bitcast(t, jnp.uint32) < jnp.uint32(WWIN)
                    off = plsc.cumsum(jnp.where(hit, 1, 0))
                    pos = cnt + off - 1
                    row = jnp.bitwise_and(
                        lax.shift_right_logical(pos, 7), RC - 1)
                    col = jnp.bitwise_and(pos, RW - 1)
                    plsc.store_scatter(cidx, [row, col], t, mask=hit)
                    plsc.store_scatter(cval, [row, col], uv, mask=hit)
                    cnt = cnt + plsc.all_reduce_population_count(hit)
                return cnt

            cnt = lax.fori_loop(0, ROWS, one_row, cnt)
            full = lax.shift_right_logical(jnp.max(cnt), 7)

            def flush_row(s, w):
                r = jnp.bitwise_and(s, RC - 1)
                pltpu.async_copy(cval.at[r], win_ref.at[cidx.at[r]], sems,
                                 add=True)

                @pl.when(s - w >= MAXOUT)
                def _():
                    wait_scat()

                return jnp.where(s - w >= MAXOUT, w + 1, w)

            waited = lax.fori_loop(flushed, full, flush_row, waited)
            return cnt, full, waited

        def two_chunks(g, carry):
            cnt, flushed, waited = carry
            wait(i0, v0, sem0)
            cnt, flushed, waited = process(i0, v0, cnt, flushed, waited)

            @pl.when(g < NCHUNK // 2 - 1)
            def _():
                fire(2 * g + 2, i0, v0, sem0)

            wait(i1, v1, sem1)
            cnt, flushed, waited = process(i1, v1, cnt, flushed, waited)

            @pl.when(g < NCHUNK // 2 - 1)
            def _():
                fire(2 * g + 3, i1, v1, sem1)

            return cnt, flushed, waited

        cnt, flushed, waited = lax.fori_loop(
            0, NCHUNK // 2, two_chunks,
            (jnp.zeros((16,), jnp.int32), jnp.int32(0), jnp.int32(0)))

        def dwait(s, _):
            wait_scat()
            return 0

        lax.fori_loop(waited, flushed, dwait, 0)

        cnt_s = jnp.max(cnt)
        q = jnp.bitwise_and(cnt_s, RW - 1)
        prow = jnp.bitwise_and(lax.shift_right_logical(cnt_s, 7), RC - 1)

        @pl.when(q > 0)
        def _():
            for j in range(RW // 16):
                slc = pl.ds(j * 16, 16)
                keep = (iota16 + j * 16) < q
                cval[prow, slc] = jnp.where(keep, cval[prow, slc], 0.0)
                cidx[prow, slc] = jnp.where(keep, cidx[prow, slc],
                                            (iota16 + j * 16) * 52)
            pltpu.sync_copy(cval.at[prow], win_ref.at[cidx.at[prow]],
                            add=True)

        plsc.subcore_barrier()

        pltpu.sync_copy(
            win_ref.at[pl.ds(sid * FLUSH, FLUSH)],
            out_hbm.at[pl.ds(base + sid * FLUSH, FLUSH)],
        )
        plsc.subcore_barrier()
        return 0

    lax.fori_loop(0, NP, one_pass, 0)


@jax.jit
def kernel(updates, mask):
    flat_idx = jnp.reshape(mask, (-1,)).astype(jnp.int32)
    flat_upd = jnp.reshape(updates, (-1,))
    mesh = plsc.VectorSubcoreMesh(core_axis_name="c", subcore_axis_name="s")
    out = pl.kernel(
        _body,
        compiler_params=pltpu.CompilerParams(needs_layout_passes=False),
        out_type=jax.ShapeDtypeStruct((N,), jnp.float32),
        mesh=mesh,
        scratch_types=[
            pltpu.VMEM_SHARED((WWIN,), jnp.float32),
            pltpu.VMEM((CHUNK,), jnp.int32),
            pltpu.VMEM((CHUNK,), jnp.float32),
            pltpu.VMEM((CHUNK,), jnp.int32),
            pltpu.VMEM((CHUNK,), jnp.float32),
            pltpu.VMEM((RC, RW), jnp.int32),
            pltpu.VMEM((RC, RW), jnp.float32),
            pltpu.VMEM((ZCH,), jnp.float32),
            pltpu.SemaphoreType.DMA,
            pltpu.SemaphoreType.DMA,
            pltpu.SemaphoreType.DMA,
            pltpu.SemaphoreType.DMA,
        ],
    )(flat_idx, flat_upd)
    return jnp.reshape(out, (B, H * 2, W_IN * 2, C_CH))


# speedup vs baseline: 4.0547x; 1.0625x over previous
import jax
import jax.numpy as jnp
from jax import lax
from jax.experimental import pallas as pl
from jax.experimental.pallas import tpu as pltpu
from jax.experimental.pallas import tpu_sc as plsc

B, H, W_IN, C_CH = 8, 112, 112, 96
M = B * H * W_IN * C_CH
N = M * 4
NC, NS = 2, 16
NP = 14
WWIN = N // (NP * NC)
SEG = M // NS
CHUNK = 6144
NCHUNK = SEG // CHUNK
ROWS = CHUNK // 128
RW = 128
RC = 64
MAXOUT = 8
FLUSH = WWIN // NS
ZCH = 2048
NZ = FLUSH // ZCH


def _body(idx_hbm, upd_hbm, out_hbm, win_ref, i0, v0, i1, v1, cidx, cval,
          zbuf, sem0, sem1, semz, sems):
    cid = lax.axis_index("c")
    sid = lax.axis_index("s")
    zeros16 = jnp.zeros((16,), jnp.float32)
    iota16 = lax.iota(jnp.int32, 16)

    def zero_zbuf(i, _):
        zbuf[pl.ds(i * 16, 16)] = zeros16
        return 0

    lax.fori_loop(0, ZCH // 16, zero_zbuf, 0)

    def fire(ch, ib, vb, sem):
        off = sid * SEG + ch * CHUNK
        pltpu.async_copy(idx_hbm.at[pl.ds(off, CHUNK)], ib, sem)
        pltpu.async_copy(upd_hbm.at[pl.ds(off, CHUNK)], vb, sem)

    def wait(ib, vb, sem):
        pltpu.make_async_copy(idx_hbm.at[pl.ds(0, CHUNK)], ib, sem).wait()
        pltpu.make_async_copy(upd_hbm.at[pl.ds(0, CHUNK)], vb, sem).wait()

    def wait_scat():
        pltpu.make_async_copy(cval.at[0], win_ref.at[cidx.at[0]],
                              sems).wait()

    def one_pass(p, _):
        win = p * NC + cid
        base = win * WWIN

        fire(0, i0, v0, sem0)
        fire(1, i1, v1, sem1)

        def zfill(z, _):
            pltpu.async_copy(
                zbuf, win_ref.at[pl.ds(sid * FLUSH + z * ZCH, ZCH)], semz)
            return 0

        lax.fori_loop(0, NZ, zfill, 0)

        def zwait(z, _):
            pltpu.make_async_copy(
                zbuf, win_ref.at[pl.ds(sid * FLUSH, ZCH)], semz).wait()
            return 0

        lax.fori_loop(0, NZ, zwait, 0)
        plsc.subcore_barrier()

        def process(ib, vb, cnt, flushed, waited):
            def one_row(r, cnt):
                for c in range(8):
                    sl = pl.ds(r * 128 + c * 16, 16)
                    t = ib[sl] - base
                    uv = vb[sl]
                    hit = plsc.bitcast(t, jnp.uint32) < jnp.uint32(WWIN)
                    off = plsc.cumsum(jnp.where(hit, 1, 0))
                    pos = cnt + off - 1
                    row = jnp.bitwise_and(
                        lax.shift_right_logical(pos, 7), RC - 1)
                    col = jnp.bitwise_and(pos, RW - 1)
                    plsc.store_scatter(cidx, [row, col], t, mask=hit)
                    plsc.store_scatter(cval, [row, col], uv, mask=hit)
                    cnt = cnt + plsc.all_reduce_population_count(hit)
                return cnt

            cnt = plsc.parallel_loop(0, ROWS, 1, unroll=2, carry=cnt)(one_row)
            full = lax.shift_right_logical(jnp.max(cnt), 7)

            def flush_row(s, w):
                r = jnp.bitwise_and(s, RC - 1)
                pltpu.async_copy(cval.at[r], win_ref.at[cidx.at[r]], sems,
                                 add=True)

                @pl.when(s - w >= MAXOUT)
                def _():
                    wait_scat()

                return jnp.where(s - w >= MAXOUT, w + 1, w)

            waited = lax.fori_loop(flushed, full, flush_row, waited)
            return cnt, full, waited

        def two_chunks(g, carry):
            cnt, flushed, waited = carry
            wait(i0, v0, sem0)
            cnt, flushed, waited = process(i0, v0, cnt, flushed, waited)

            @pl.when(g < NCHUNK // 2 - 1)
            def _():
                fire(2 * g + 2, i0, v0, sem0)

            wait(i1, v1, sem1)
            cnt, flushed, waited = process(i1, v1, cnt, flushed, waited)

            @pl.when(g < NCHUNK // 2 - 1)
            def _():
                fire(2 * g + 3, i1, v1, sem1)

            return cnt, flushed, waited

        cnt, flushed, waited = lax.fori_loop(
            0, NCHUNK // 2, two_chunks,
            (jnp.zeros((16,), jnp.int32), jnp.int32(0), jnp.int32(0)))

        def dwait(s, _):
            wait_scat()
            return 0

        lax.fori_loop(waited, flushed, dwait, 0)

        cnt_s = jnp.max(cnt)
        q = jnp.bitwise_and(cnt_s, RW - 1)
        prow = jnp.bitwise_and(lax.shift_right_logical(cnt_s, 7), RC - 1)

        @pl.when(q > 0)
        def _():
            for j in range(RW // 16):
                slc = pl.ds(j * 16, 16)
                keep = (iota16 + j * 16) < q
                cval[prow, slc] = jnp.where(keep, cval[prow, slc], 0.0)
                cidx[prow, slc] = jnp.where(keep, cidx[prow, slc],
                                            (iota16 + j * 16) * 52)
            pltpu.sync_copy(cval.at[prow], win_ref.at[cidx.at[prow]],
                            add=True)

        plsc.subcore_barrier()

        pltpu.sync_copy(
            win_ref.at[pl.ds(sid * FLUSH, FLUSH)],
            out_hbm.at[pl.ds(base + sid * FLUSH, FLUSH)],
        )
        plsc.subcore_barrier()
        return 0

    lax.fori_loop(0, NP, one_pass, 0)


@jax.jit
def kernel(updates, mask):
    flat_idx = jnp.reshape(mask, (-1,)).astype(jnp.int32)
    flat_upd = jnp.reshape(updates, (-1,))
    mesh = plsc.VectorSubcoreMesh(core_axis_name="c", subcore_axis_name="s")
    out = pl.kernel(
        _body,
        compiler_params=pltpu.CompilerParams(needs_layout_passes=False),
        out_type=jax.ShapeDtypeStruct((N,), jnp.float32),
        mesh=mesh,
        scratch_types=[
            pltpu.VMEM_SHARED((WWIN,), jnp.float32),
            pltpu.VMEM((CHUNK,), jnp.int32),
            pltpu.VMEM((CHUNK,), jnp.float32),
            pltpu.VMEM((CHUNK,), jnp.int32),
            pltpu.VMEM((CHUNK,), jnp.float32),
            pltpu.VMEM((RC, RW), jnp.int32),
            pltpu.VMEM((RC, RW), jnp.float32),
            pltpu.VMEM((ZCH,), jnp.float32),
            pltpu.SemaphoreType.DMA,
            pltpu.SemaphoreType.DMA,
            pltpu.SemaphoreType.DMA,
            pltpu.SemaphoreType.DMA,
        ],
    )(flat_idx, flat_upd)
    return jnp.reshape(out, (B, H * 2, W_IN * 2, C_CH))


# speedup vs baseline: 4.1744x; 1.0295x over previous
import jax
import jax.numpy as jnp
from jax import lax
from jax.experimental import pallas as pl
from jax.experimental.pallas import tpu as pltpu
from jax.experimental.pallas import tpu_sc as plsc

B, H, W_IN, C_CH = 8, 112, 112, 96
M = B * H * W_IN * C_CH
N = M * 4
NC, NS = 2, 16
NP = 14
WWIN = N // (NP * NC)
SEG = M // NS
CHUNK = 6144
NCHUNK = SEG // CHUNK
ROWS = CHUNK // 128
RW = 128
RC = 64
MAXOUT = 8
FLUSH = WWIN // NS
ZCH = 2048
NZ = FLUSH // ZCH


def _body(idx_hbm, upd_hbm, out_hbm, win_ref, i0, v0, i1, v1, cidx, cval,
          zbuf, sem0, sem1, semz, sems):
    cid = lax.axis_index("c")
    sid = lax.axis_index("s")
    zeros16 = jnp.zeros((16,), jnp.float32)
    iota16 = lax.iota(jnp.int32, 16)

    def zero_zbuf(i, _):
        zbuf[pl.ds(i * 16, 16)] = zeros16
        return 0

    lax.fori_loop(0, ZCH // 16, zero_zbuf, 0)

    def fire(ch, ib, vb, sem):
        off = sid * SEG + ch * CHUNK
        pltpu.async_copy(idx_hbm.at[pl.ds(off, CHUNK)], ib, sem)
        pltpu.async_copy(upd_hbm.at[pl.ds(off, CHUNK)], vb, sem)

    def wait(ib, vb, sem):
        pltpu.make_async_copy(idx_hbm.at[pl.ds(0, CHUNK)], ib, sem).wait()
        pltpu.make_async_copy(upd_hbm.at[pl.ds(0, CHUNK)], vb, sem).wait()

    def wait_scat():
        pltpu.make_async_copy(cval.at[0], win_ref.at[cidx.at[0]],
                              sems).wait()

    def one_pass(p, _):
        win = p * NC + cid
        base = win * WWIN

        fire(0, i0, v0, sem0)
        fire(1, i1, v1, sem1)

        def zfill(z, _):
            pltpu.async_copy(
                zbuf, win_ref.at[pl.ds(sid * FLUSH + z * ZCH, ZCH)], semz)
            return 0

        lax.fori_loop(0, NZ, zfill, 0)

        def zwait(z, _):
            pltpu.make_async_copy(
                zbuf, win_ref.at[pl.ds(sid * FLUSH, ZCH)], semz).wait()
            return 0

        lax.fori_loop(0, NZ, zwait, 0)
        plsc.subcore_barrier()

        def process(ib, vb, cnt, flushed, waited):
            def one_row(r, cnt):
                for c in range(8):
                    sl = pl.ds(r * 128 + c * 16, 16)
                    t = ib[sl] - base
                    uv = vb[sl]
                    hit = plsc.bitcast(t, jnp.uint32) < jnp.uint32(WWIN)
                    off = plsc.cumsum(jnp.where(hit, 1, 0))
                    pos = cnt + off - 1
                    row = jnp.bitwise_and(
                        lax.shift_right_logical(pos, 7), RC - 1)
                    col = jnp.bitwise_and(pos, RW - 1)
                    plsc.store_scatter(cidx, [row, col], t, mask=hit)
                    plsc.store_scatter(cval, [row, col], uv, mask=hit)
                    cnt = cnt + plsc.all_reduce_population_count(hit)
                return cnt

            cnt = plsc.parallel_loop(0, ROWS, 1, unroll=4, carry=cnt)(one_row)
            full = lax.shift_right_logical(jnp.max(cnt), 7)

            def flush_row(s, w):
                r = jnp.bitwise_and(s, RC - 1)
                pltpu.async_copy(cval.at[r], win_ref.at[cidx.at[r]], sems,
                                 add=True)

                @pl.when(s - w >= MAXOUT)
                def _():
                    wait_scat()

                return jnp.where(s - w >= MAXOUT, w + 1, w)

            waited = lax.fori_loop(flushed, full, flush_row, waited)
            return cnt, full, waited

        def two_chunks(g, carry):
            cnt, flushed, waited = carry
            wait(i0, v0, sem0)
            cnt, flushed, waited = process(i0, v0, cnt, flushed, waited)

            @pl.when(g < NCHUNK // 2 - 1)
            def _():
                fire(2 * g + 2, i0, v0, sem0)

            wait(i1, v1, sem1)
            cnt, flushed, waited = process(i1, v1, cnt, flushed, waited)

            @pl.when(g < NCHUNK // 2 - 1)
            def _():
                fire(2 * g + 3, i1, v1, sem1)

            return cnt, flushed, waited

        cnt, flushed, waited = lax.fori_loop(
            0, NCHUNK // 2, two_chunks,
            (jnp.zeros((16,), jnp.int32), jnp.int32(0), jnp.int32(0)))

        def dwait(s, _):
            wait_scat()
            return 0

        lax.fori_loop(waited, flushed, dwait, 0)

        cnt_s = jnp.max(cnt)
        q = jnp.bitwise_and(cnt_s, RW - 1)
        prow = jnp.bitwise_and(lax.shift_right_logical(cnt_s, 7), RC - 1)

        @pl.when(q > 0)
        def _():
            for j in range(RW // 16):
                slc = pl.ds(j * 16, 16)
                keep = (iota16 + j * 16) < q
                cval[prow, slc] = jnp.where(keep, cval[prow, slc], 0.0)
                cidx[prow, slc] = jnp.where(keep, cidx[prow, slc],
                                            (iota16 + j * 16) * 52)
            pltpu.sync_copy(cval.at[prow], win_ref.at[cidx.at[prow]],
                            add=True)

        plsc.subcore_barrier()

        pltpu.sync_copy(
            win_ref.at[pl.ds(sid * FLUSH, FLUSH)],
            out_hbm.at[pl.ds(base + sid * FLUSH, FLUSH)],
        )
        plsc.subcore_barrier()
        return 0

    lax.fori_loop(0, NP, one_pass, 0)


@jax.jit
def kernel(updates, mask):
    flat_idx = jnp.reshape(mask, (-1,)).astype(jnp.int32)
    flat_upd = jnp.reshape(updates, (-1,))
    mesh = plsc.VectorSubcoreMesh(core_axis_name="c", subcore_axis_name="s")
    out = pl.kernel(
        _body,
        compiler_params=pltpu.CompilerParams(needs_layout_passes=False),
        out_type=jax.ShapeDtypeStruct((N,), jnp.float32),
        mesh=mesh,
        scratch_types=[
            pltpu.VMEM_SHARED((WWIN,), jnp.float32),
            pltpu.VMEM((CHUNK,), jnp.int32),
            pltpu.VMEM((CHUNK,), jnp.float32),
            pltpu.VMEM((CHUNK,), jnp.int32),
            pltpu.VMEM((CHUNK,), jnp.float32),
            pltpu.VMEM((RC, RW), jnp.int32),
            pltpu.VMEM((RC, RW), jnp.float32),
            pltpu.VMEM((ZCH,), jnp.float32),
            pltpu.SemaphoreType.DMA,
            pltpu.SemaphoreType.DMA,
            pltpu.SemaphoreType.DMA,
            pltpu.SemaphoreType.DMA,
        ],
    )(flat_idx, flat_upd)
    return jnp.reshape(out, (B, H * 2, W_IN * 2, C_CH))
